# baseline (device time: 26400 ns/iter reference)
import jax
import jax.numpy as jnp
from jax import lax
from jax.experimental import pallas as pl
from jax.experimental.pallas import tpu as pltpu

N_DEV = 16
BLK = 128
N_GROUPS = 4
GROUP = N_DEV // N_GROUPS

_ANY = getattr(pltpu, "ANY", None)
if _ANY is None:
    _ms = getattr(pltpu, "MemorySpace", None) or getattr(
        pltpu, "TPUMemorySpace", None
    )
    _ANY = getattr(_ms, "ANY", None)
if _ANY is None:
    _ANY = pl.ANY


def kernel(x, w_mat):
    k, m_per = x.shape
    _, n = w_mat.shape

    def body(
        x_hbm,
        w_hbm,
        out_hbm,
        x_ref,
        w_ref,
        xrow_ref,
        obuf_ref,
        local_sems,
        send_sems,
        recv_sems,
    ):
        me = lax.axis_index("i")

        cp_x = pltpu.make_async_copy(x_hbm, x_ref, local_sems.at[0])
        cp_x.start()
        cp_w = pltpu.make_async_copy(w_hbm, w_ref, local_sems.at[1])
        cp_w.start()

        barrier_sem = pltpu.get_barrier_semaphore()
        for s in range(1, N_DEV):
            peer = lax.rem(me + s, N_DEV)
            pl.semaphore_signal(
                barrier_sem,
                inc=1,
                device_id=(peer,),
                device_id_type=pl.DeviceIdType.MESH,
            )
        pl.semaphore_wait(barrier_sem, N_DEV - 1)
        cp_x.wait()

        rdmas = []
        for s in range(1, N_DEV):
            d = lax.rem(me + s, N_DEV)
            rdma = pltpu.make_async_remote_copy(
                src_ref=x_ref.at[pl.ds(d * BLK, BLK)],
                dst_ref=xrow_ref.at[:, pl.ds(me * BLK, BLK)],
                send_sem=send_sems.at[s],
                recv_sem=recv_sems.at[me],
                device_id=(d,),
                device_id_type=pl.DeviceIdType.MESH,
            )
            rdma.start()
            rdmas.append(rdma)

        xrow_ref[:, pl.ds(me * BLK, BLK)] = x_ref[pl.ds(me * BLK, BLK), :]

        cp_w.wait()

        acc = jnp.zeros((m_per, n), dtype=jnp.float32)
        for g in range(N_GROUPS):
            for j in range(g * GROUP, (g + 1) * GROUP):
                recv = pltpu.make_async_remote_copy(
                    src_ref=x_ref.at[pl.ds(0, BLK)],
                    dst_ref=xrow_ref.at[:, pl.ds(j * BLK, BLK)],
                    send_sem=send_sems.at[0],
                    recv_sem=recv_sems.at[j],
                    device_id=(0,),
                    device_id_type=pl.DeviceIdType.MESH,
                )

                @pl.when(j != me)
                def _():
                    recv.wait_recv()

            acc = acc + jnp.dot(
                xrow_ref[:, pl.ds(g * GROUP * BLK, GROUP * BLK)],
                w_ref[pl.ds(g * GROUP * BLK, GROUP * BLK), :],
                preferred_element_type=jnp.float32,
            )

        c = 0.7978845608028654
        obuf_ref[:, :] = 0.5 * acc * (
            1.0 + jnp.tanh(c * (acc + 0.044715 * acc * acc * acc))
        )
        cp_out = pltpu.make_async_copy(obuf_ref, out_hbm, local_sems.at[2])
        cp_out.start()
        cp_out.wait()

        for r in rdmas:
            r.wait_send()

    return pl.pallas_call(
        body,
        out_shape=jax.ShapeDtypeStruct((m_per, n), jnp.float32),
        in_specs=[
            pl.BlockSpec(memory_space=_ANY),
            pl.BlockSpec(memory_space=_ANY),
        ],
        out_specs=pl.BlockSpec(memory_space=_ANY),
        scratch_shapes=[
            pltpu.VMEM((k, m_per), jnp.float32),
            pltpu.VMEM((k, n), jnp.float32),
            pltpu.VMEM((m_per, k), jnp.float32),
            pltpu.VMEM((m_per, n), jnp.float32),
            pltpu.SemaphoreType.DMA((3,)),
            pltpu.SemaphoreType.DMA((N_DEV,)),
            pltpu.SemaphoreType.DMA((N_DEV,)),
        ],
        compiler_params=pltpu.CompilerParams(collective_id=0),
    )(x, w_mat)


# device time: 23802 ns/iter; 1.1092x vs baseline; 1.1092x over previous
import jax
import jax.numpy as jnp
from jax import lax
from jax.experimental import pallas as pl
from jax.experimental.pallas import tpu as pltpu

N_DEV = 16
BLK = 128
N_GROUPS = 4
GROUP = N_DEV // N_GROUPS

_ANY = pltpu.MemorySpace.HBM


def kernel(x, w_mat):
    k, m_per = x.shape
    _, n = w_mat.shape

    def body(
        x_hbm,
        w_hbm,
        out_hbm,
        x_ref,
        w_ref,
        xrow_ref,
        obuf_ref,
        local_sems,
        send_sems,
        recv_sems,
    ):
        me = lax.axis_index("i")

        cp_x = pltpu.make_async_copy(x_hbm, x_ref, local_sems.at[0])
        cp_x.start()
        cp_w = pltpu.make_async_copy(w_hbm, w_ref, local_sems.at[1])
        cp_w.start()

        barrier_sem = pltpu.get_barrier_semaphore()
        for s in range(1, N_DEV):
            peer = lax.rem(me + s, N_DEV)
            pl.semaphore_signal(
                barrier_sem,
                inc=1,
                device_id=(peer,),
                device_id_type=pl.DeviceIdType.MESH,
            )
        pl.semaphore_wait(barrier_sem, N_DEV - 1)
        cp_x.wait()

        rdmas = []
        for s in range(1, N_DEV):
            d = lax.rem(me + s, N_DEV)
            rdma = pltpu.make_async_remote_copy(
                src_ref=x_ref.at[pl.ds(d * BLK, BLK)],
                dst_ref=xrow_ref.at[:, pl.ds(me * BLK, BLK)],
                send_sem=send_sems.at[s],
                recv_sem=recv_sems.at[me],
                device_id=(d,),
                device_id_type=pl.DeviceIdType.MESH,
            )
            rdma.start()
            rdmas.append(rdma)

        xrow_ref[:, pl.ds(me * BLK, BLK)] = x_ref[pl.ds(me * BLK, BLK), :]

        cp_w.wait()

        acc = jnp.zeros((m_per, n), dtype=jnp.float32)
        for g in range(N_GROUPS):
            for j in range(g * GROUP, (g + 1) * GROUP):
                recv = pltpu.make_async_remote_copy(
                    src_ref=x_ref.at[pl.ds(0, BLK)],
                    dst_ref=xrow_ref.at[:, pl.ds(j * BLK, BLK)],
                    send_sem=send_sems.at[0],
                    recv_sem=recv_sems.at[j],
                    device_id=(0,),
                    device_id_type=pl.DeviceIdType.MESH,
                )

                @pl.when(j != me)
                def _():
                    recv.wait_recv()

            acc = acc + jnp.dot(
                xrow_ref[:, pl.ds(g * GROUP * BLK, GROUP * BLK)],
                w_ref[pl.ds(g * GROUP * BLK, GROUP * BLK), :],
                preferred_element_type=jnp.float32,
            )

        c = 0.7978845608028654
        obuf_ref[:, :] = 0.5 * acc * (
            1.0 + jnp.tanh(c * (acc + 0.044715 * acc * acc * acc))
        )
        cp_out = pltpu.make_async_copy(obuf_ref, out_hbm, local_sems.at[2])
        cp_out.start()
        cp_out.wait()

        for r in rdmas:
            r.wait_send()

    return pl.pallas_call(
        body,
        out_shape=jax.ShapeDtypeStruct((m_per, n), jnp.float32),
        in_specs=[
            pl.BlockSpec(memory_space=_ANY),
            pl.BlockSpec(memory_space=_ANY),
        ],
        out_specs=pl.BlockSpec(memory_space=_ANY),
        scratch_shapes=[
            pltpu.VMEM((k, m_per), jnp.float32),
            pltpu.VMEM((k, n), jnp.float32),
            pltpu.VMEM((m_per, k), jnp.float32),
            pltpu.VMEM((m_per, n), jnp.float32),
            pltpu.SemaphoreType.DMA((3,)),
            pltpu.SemaphoreType.DMA((N_DEV,)),
            pltpu.SemaphoreType.DMA((N_DEV,)),
        ],
        compiler_params=pltpu.CompilerParams(collective_id=0),
    )(x, w_mat)


# device time: 18099 ns/iter; 1.4586x vs baseline; 1.3151x over previous
import jax
import jax.numpy as jnp
from jax import lax
from jax.experimental import pallas as pl
from jax.experimental.pallas import tpu as pltpu

N_DEV = 16
BLK = 128
N_GROUPS = 4
GROUP = N_DEV // N_GROUPS

_ANY = pltpu.MemorySpace.HBM


def kernel(x, w_mat):
    k, m_per = x.shape
    _, n = w_mat.shape

    def body(
        x_hbm,
        w_hbm,
        out_hbm,
        x_ref,
        w_ref,
        xrow_ref,
        obuf_ref,
        local_sems,
        send_sems,
        recv_sems,
    ):
        me = lax.axis_index("i")

        cp_x = pltpu.make_async_copy(x_hbm, x_ref, local_sems.at[0])
        cp_x.start()
        cp_w = pltpu.make_async_copy(w_hbm, w_ref, local_sems.at[1])
        cp_w.start()

        barrier_sem = pltpu.get_barrier_semaphore()
        for s in range(1, N_DEV):
            peer = lax.rem(me + s, N_DEV)
            pl.semaphore_signal(
                barrier_sem,
                inc=1,
                device_id=(peer,),
                device_id_type=pl.DeviceIdType.MESH,
            )
        pl.semaphore_wait(barrier_sem, N_DEV - 1)
        cp_x.wait()

        rdmas = []
        for s in range(1, N_DEV):
            d = lax.rem(me + s, N_DEV)
            rdma = pltpu.make_async_remote_copy(
                src_ref=x_ref.at[pl.ds(d * BLK, BLK)],
                dst_ref=xrow_ref.at[:, pl.ds(me * BLK, BLK)],
                send_sem=send_sems.at[s],
                recv_sem=recv_sems.at[me],
                device_id=(d,),
                device_id_type=pl.DeviceIdType.MESH,
            )
            rdma.start()
            rdmas.append(rdma)

        xrow_ref[:, pl.ds(me * BLK, BLK)] = x_ref[pl.ds(me * BLK, BLK), :]

        cp_w.wait()

        acc = jnp.zeros((m_per, n), dtype=jnp.float32)
        for g in range(N_GROUPS):
            for j in range(g * GROUP, (g + 1) * GROUP):
                recv = pltpu.make_async_remote_copy(
                    src_ref=x_ref.at[pl.ds(0, BLK)],
                    dst_ref=xrow_ref.at[:, pl.ds(j * BLK, BLK)],
                    send_sem=send_sems.at[0],
                    recv_sem=recv_sems.at[j],
                    device_id=(0,),
                    device_id_type=pl.DeviceIdType.MESH,
                )

                @pl.when(j != me)
                def _():
                    recv.wait_recv()

            acc = acc + jnp.dot(
                xrow_ref[:, pl.ds(g * GROUP * BLK, GROUP * BLK)],
                w_ref[pl.ds(g * GROUP * BLK, GROUP * BLK), :],
                preferred_element_type=jnp.float32,
            )

        c = 0.7978845608028654
        obuf_ref[:, :] = 0.5 * acc * (
            1.0 + jnp.tanh(c * (acc + 0.044715 * acc * acc * acc))
        )
        cp_out = pltpu.make_async_copy(obuf_ref, out_hbm, local_sems.at[2])
        cp_out.start()
        cp_out.wait()

        for r in rdmas:
            r.wait_send()

    x = pltpu.with_memory_space_constraint(x, _ANY)
    w_mat = pltpu.with_memory_space_constraint(w_mat, _ANY)
    return pl.pallas_call(
        body,
        out_shape=jax.ShapeDtypeStruct((m_per, n), jnp.float32),
        in_specs=[
            pl.BlockSpec(memory_space=_ANY),
            pl.BlockSpec(memory_space=_ANY),
        ],
        out_specs=pl.BlockSpec(memory_space=_ANY),
        scratch_shapes=[
            pltpu.VMEM((k, m_per), jnp.float32),
            pltpu.VMEM((k, n), jnp.float32),
            pltpu.VMEM((m_per, k), jnp.float32),
            pltpu.VMEM((m_per, n), jnp.float32),
            pltpu.SemaphoreType.DMA((3,)),
            pltpu.SemaphoreType.DMA((N_DEV,)),
            pltpu.SemaphoreType.DMA((N_DEV,)),
        ],
        compiler_params=pltpu.CompilerParams(collective_id=0),
    )(x, w_mat)


# device time: 12442 ns/iter; 2.1218x vs baseline; 1.4547x over previous
import jax
import jax.numpy as jnp
from jax import lax
from jax.experimental import pallas as pl
from jax.experimental.pallas import tpu as pltpu

N_DEV = 16
BLK = 128
N_GROUPS = 4
GROUP = N_DEV // N_GROUPS

_ANY = pltpu.MemorySpace.HBM


def kernel(x, w_mat):
    k, m_per = x.shape
    _, n = w_mat.shape

    def body(
        x_hbm,
        w_hbm,
        out_hbm,
        x_ref,
        xbf_ref,
        w_ref,
        xrow_ref,
        obuf_ref,
        local_sems,
        send_sems,
        recv_sems,
    ):
        me = lax.axis_index("i")

        cp_x = pltpu.make_async_copy(x_hbm, x_ref, local_sems.at[0])
        cp_x.start()
        cp_w = pltpu.make_async_copy(w_hbm, w_ref, local_sems.at[1])
        cp_w.start()

        barrier_sem = pltpu.get_barrier_semaphore()
        for s in range(1, N_DEV):
            peer = lax.rem(me + s, N_DEV)
            pl.semaphore_signal(
                barrier_sem,
                inc=1,
                device_id=(peer,),
                device_id_type=pl.DeviceIdType.MESH,
            )
        pl.semaphore_wait(barrier_sem, N_DEV - 1)
        cp_x.wait()

        xbf_ref[:, :] = x_ref[:, :].astype(jnp.bfloat16)

        rdmas = []
        for s in range(1, N_DEV):
            d = lax.rem(me + s, N_DEV)
            rdma = pltpu.make_async_remote_copy(
                src_ref=xbf_ref.at[pl.ds(d * BLK, BLK)],
                dst_ref=xrow_ref.at[:, pl.ds(me * BLK, BLK)],
                send_sem=send_sems.at[s],
                recv_sem=recv_sems.at[me],
                device_id=(d,),
                device_id_type=pl.DeviceIdType.MESH,
            )
            rdma.start()
            rdmas.append(rdma)

        xrow_ref[:, pl.ds(me * BLK, BLK)] = xbf_ref[pl.ds(me * BLK, BLK), :]

        cp_w.wait()

        acc = jnp.zeros((m_per, n), dtype=jnp.float32)
        for g in range(N_GROUPS):
            for j in range(g * GROUP, (g + 1) * GROUP):
                recv = pltpu.make_async_remote_copy(
                    src_ref=xbf_ref.at[pl.ds(0, BLK)],
                    dst_ref=xrow_ref.at[:, pl.ds(j * BLK, BLK)],
                    send_sem=send_sems.at[0],
                    recv_sem=recv_sems.at[j],
                    device_id=(0,),
                    device_id_type=pl.DeviceIdType.MESH,
                )

                @pl.when(j != me)
                def _():
                    recv.wait_recv()

            acc = acc + jnp.dot(
                xrow_ref[:, pl.ds(g * GROUP * BLK, GROUP * BLK)],
                w_ref[pl.ds(g * GROUP * BLK, GROUP * BLK), :],
                preferred_element_type=jnp.float32,
            )

        c = 0.7978845608028654
        obuf_ref[:, :] = 0.5 * acc * (
            1.0 + jnp.tanh(c * (acc + 0.044715 * acc * acc * acc))
        )
        cp_out = pltpu.make_async_copy(obuf_ref, out_hbm, local_sems.at[2])
        cp_out.start()
        cp_out.wait()

        for r in rdmas:
            r.wait_send()

    x = pltpu.with_memory_space_constraint(x, _ANY)
    w_mat = pltpu.with_memory_space_constraint(w_mat, _ANY)
    return pl.pallas_call(
        body,
        out_shape=jax.ShapeDtypeStruct((m_per, n), jnp.float32),
        in_specs=[
            pl.BlockSpec(memory_space=_ANY),
            pl.BlockSpec(memory_space=_ANY),
        ],
        out_specs=pl.BlockSpec(memory_space=_ANY),
        scratch_shapes=[
            pltpu.VMEM((k, m_per), jnp.float32),
            pltpu.VMEM((k, m_per), jnp.bfloat16),
            pltpu.VMEM((k, n), jnp.float32),
            pltpu.VMEM((m_per, k), jnp.bfloat16),
            pltpu.VMEM((m_per, n), jnp.float32),
            pltpu.SemaphoreType.DMA((3,)),
            pltpu.SemaphoreType.DMA((N_DEV,)),
            pltpu.SemaphoreType.DMA((N_DEV,)),
        ],
        compiler_params=pltpu.CompilerParams(collective_id=0),
    )(x, w_mat)
